# baseline (speedup 1.0000x reference)
import functools

import jax
import jax.numpy as jnp
from jax import lax
from jax.experimental import pallas as pl
from jax.experimental.pallas import tpu as pltpu
from jax.experimental.pallas import tpu_sc as plsc

ROWS = 1024
IN_C = 32 + 1024 + 32768
OUT_C = 1 + IN_C
LEVELS = ((0, 32), (32, 1024), (1056, 32768))
NC = 2
NS = 16
NW = NC * NS
ROWS_PER_W = ROWS // NW
L = 16

_LN2 = 0.6931471805599453


def _vlog(s):
    bits = lax.bitcast_convert_type(s, jnp.int32)
    e = jnp.float32(1.0) * (lax.shift_right_logical(bits, 23) - 127)
    m = lax.bitcast_convert_type(
        (bits & jnp.int32(0x007FFFFF)) | jnp.int32(0x3F800000), jnp.float32)
    t = (m - 1.0) / (m + 1.0)
    t2 = t * t
    p = 1.0 + t2 * (jnp.float32(1 / 3) + t2 * (jnp.float32(1 / 5)
        + t2 * (jnp.float32(1 / 7) + t2 * jnp.float32(1 / 9))))
    return e * jnp.float32(_LN2) + 2.0 * t * p


def _lane_allreduce(v, op):
    lane = lax.iota(jnp.int32, L)
    for sh in (8, 4, 2, 1):
        v = op(v, v.at[lane ^ sh].get(mode="promise_in_bounds"))
    return v


def _levelwise_body(scores_hbm, out_hbm, inbuf, outbuf):
    wid = lax.axis_index("s") * NC + lax.axis_index("c")
    lane = lax.iota(jnp.int32, L)

    def do_row(i, _):
        row = wid * ROWS_PER_W + i
        pltpu.sync_copy(scores_hbm.at[row], inbuf)

        for start, size in LEVELS:
            nchunks = size // L

            def max_step(j, acc):
                return jnp.maximum(acc, inbuf[pl.ds(start + j * L, L)])
            maxv = lax.fori_loop(0, nchunks, max_step,
                                 jnp.full((L,), -jnp.inf, jnp.float32))
            m = _lane_allreduce(maxv, jnp.maximum)

            def sum_step(j, acc):
                return acc + jnp.exp(inbuf[pl.ds(start + j * L, L)] - m)
            sumv = lax.fori_loop(0, nchunks, sum_step,
                                 jnp.zeros((L,), jnp.float32))
            s = _lane_allreduce(sumv, jnp.add)

            logz = m + _vlog(s)

            def sub_step(j, _):
                off = start + j * L
                outbuf[pl.ds(1 + off, L)] = inbuf[pl.ds(off, L)] - logz
                return 0
            lax.fori_loop(0, nchunks, sub_step, 0)

        head = outbuf[pl.ds(0, L)]
        outbuf[pl.ds(0, L)] = jnp.where(lane == 0, jnp.float32(0.0), head)
        pltpu.sync_copy(outbuf, out_hbm.at[row])
        return 0

    lax.fori_loop(0, ROWS_PER_W, do_row, 0)


@jax.jit
def kernel(scores):
    mesh = plsc.VectorSubcoreMesh(core_axis_name="c", subcore_axis_name="s")
    f = functools.partial(
        pl.kernel,
        mesh=mesh,
        out_type=jax.ShapeDtypeStruct((ROWS, OUT_C), jnp.float32),
        scratch_types=[
            pltpu.VMEM((IN_C,), jnp.float32),
            pltpu.VMEM((OUT_C,), jnp.float32),
        ],
    )(_levelwise_body)
    return f(scores)
